# Initial kernel scaffold; baseline (speedup 1.0000x reference)
import jax
import jax.numpy as jnp
from jax.experimental import pallas as pl


def kernel(seq_len, pos_emb):
    raise NotImplementedError("write your pallas kernel here")


# speedup vs baseline: 1.4826x; 1.4826x over previous
import jax
import jax.numpy as jnp
from jax import lax
from jax.experimental import pallas as pl
from jax.experimental.pallas import tpu as pltpu, tpu_sc as plsc

_ROWS = 8192
_DIM = 1024
_NC = 2
_NS = 16
_NW = _NC * _NS
_RPW = _ROWS // _NW
_CHUNK = 32
_NCHUNK = _RPW // _CHUNK


def _gather_body(table, idx, out, idx_v, buf0, buf1, gs0, gs1, ws0, ws1):
    wid = lax.axis_index("s") * _NC + lax.axis_index("c")
    base = wid * _RPW
    pltpu.sync_copy(idx.at[wid], idx_v)
    bufs = (buf0, buf1)
    gsems = (gs0, gs1)
    wsems = (ws0, ws1)
    g = [None] * _NCHUNK
    w = [None] * _NCHUNK
    g[0] = pltpu.async_copy(table.at[idx_v.at[0]], bufs[0], gsems[0])
    for c in range(_NCHUNK):
        b = c & 1
        g[c].wait()
        w[c] = pltpu.async_copy(
            bufs[b], out.at[pl.ds(base + c * _CHUNK, _CHUNK)], wsems[b])
        if c + 1 < _NCHUNK:
            if c >= 1:
                w[c - 1].wait()
            nb = (c + 1) & 1
            g[c + 1] = pltpu.async_copy(
                table.at[idx_v.at[c + 1]], bufs[nb], gsems[nb])
    w[_NCHUNK - 2].wait()
    w[_NCHUNK - 1].wait()


def kernel(seq_len, pos_emb):
    offset = jnp.asarray(seq_len, jnp.int32) - _ROWS
    positions = jnp.clip(
        jnp.arange(_ROWS, dtype=jnp.int32) + offset, 0, _ROWS - 1)
    idx = positions.reshape(_NW, _NCHUNK, _CHUNK)
    f = pl.kernel(
        _gather_body,
        mesh=plsc.VectorSubcoreMesh(core_axis_name="c", subcore_axis_name="s"),
        out_type=jax.ShapeDtypeStruct((_ROWS, _DIM), jnp.float32),
        scratch_types=[
            pltpu.VMEM((_NCHUNK, _CHUNK), jnp.int32),
            pltpu.VMEM((_CHUNK, _DIM), jnp.float32),
            pltpu.VMEM((_CHUNK, _DIM), jnp.float32),
            pltpu.SemaphoreType.DMA,
            pltpu.SemaphoreType.DMA,
            pltpu.SemaphoreType.DMA,
            pltpu.SemaphoreType.DMA,
        ],
    )
    return f(pos_emb, idx)


# speedup vs baseline: 1.5707x; 1.0595x over previous
import jax
import jax.numpy as jnp
from jax import lax
from jax.experimental import pallas as pl
from jax.experimental.pallas import tpu as pltpu, tpu_sc as plsc

_ROWS = 8192
_DIM = 1024
_NC = 2
_NS = 16
_NW = _NC * _NS
_RPW = _ROWS // _NW
_CHUNK = 32
_NCHUNK = _RPW // _CHUNK
_NBUF = 3


def _gather_body(table, idx, out, idx_v, *rest):
    bufs = rest[:_NBUF]
    gsems = rest[_NBUF:2 * _NBUF]
    wsems = rest[2 * _NBUF:3 * _NBUF]
    wid = lax.axis_index("s") * _NC + lax.axis_index("c")
    base = wid * _RPW
    pltpu.sync_copy(idx.at[wid], idx_v)
    g = [None] * _NCHUNK
    w = [None] * _NCHUNK
    for c in range(min(_NBUF, _NCHUNK)):
        g[c] = pltpu.async_copy(table.at[idx_v.at[c]], bufs[c], gsems[c])
    for c in range(_NCHUNK):
        b = c % _NBUF
        g[c].wait()
        w[c] = pltpu.async_copy(
            bufs[b], out.at[pl.ds(base + c * _CHUNK, _CHUNK)], wsems[b])
        n = c + _NBUF
        if n < _NCHUNK:
            w[c].wait()
            g[n] = pltpu.async_copy(table.at[idx_v.at[n]], bufs[b], gsems[b])
    for c in range(max(0, _NCHUNK - _NBUF), _NCHUNK):
        w[c].wait()


def kernel(seq_len, pos_emb):
    offset = jnp.asarray(seq_len, jnp.int32) - _ROWS
    positions = jnp.clip(
        jnp.arange(_ROWS, dtype=jnp.int32) + offset, 0, _ROWS - 1)
    idx = positions.reshape(_NW, _NCHUNK, _CHUNK)
    f = pl.kernel(
        _gather_body,
        mesh=plsc.VectorSubcoreMesh(core_axis_name="c", subcore_axis_name="s"),
        out_type=jax.ShapeDtypeStruct((_ROWS, _DIM), jnp.float32),
        scratch_types=(
            [pltpu.VMEM((_NCHUNK, _CHUNK), jnp.int32)]
            + [pltpu.VMEM((_CHUNK, _DIM), jnp.float32)] * _NBUF
            + [pltpu.SemaphoreType.DMA] * (2 * _NBUF)
        ),
    )
    return f(pos_emb, idx)
